# baseline (speedup 1.0000x reference)
import jax
import jax.numpy as jnp
from jax import lax
from jax.experimental import pallas as pl
from jax.experimental.pallas import tpu as pltpu
from jax.experimental.pallas import tpu_sc as plsc

L = 16
NC = 2
NS = 16
NW = NC * NS
B = 16384
NF = 32
NR = 1_000_000
CHUNK = 512
CPW = 61
NCH_FULL = 1953
TAIL0 = 999_936
NTAIL = NR - TAIL0
BIG = 0x40000000
BINCAP = 2048
FLUSH_AT = 112
NBUF = 3


def _scalar(splat):
    return jnp.max(splat)


def _pass(idx_v, rlist, blist, binr, binb, rowstage, bidx,
          staged, biasring, tailstage, tailbias, sems,
          idx_hbm, table_hbm, bias_hbm, tail_t_hbm, tail_b_hbm, out_int,
          wid, dump):
    lanes = lax.iota(jnp.int32, L)
    lo = wid * (CPW * CHUNK)
    hi = jnp.where(wid == NW - 1, NR, lo + CPW * CHUNK)

    pltpu.sync_copy(idx_hbm, idx_v)
    gbase = wid * CPW

    def fire(local_q, buf):
        g = jnp.minimum(gbase + local_q, NCH_FULL - 1)
        r0 = g * CHUNK
        cu = pltpu.async_copy(table_hbm.at[:, pl.ds(r0, CHUNK)],
                              staged.at[buf], sems[buf])
        cb = pltpu.async_copy(bias_hbm.at[:, pl.ds(r0, CHUNK)],
                              biasring.at[buf], sems[buf])
        return cu, cb

    def wait(buf):
        pltpu.make_async_copy(table_hbm.at[:, pl.ds(0, CHUNK)],
                              staged.at[buf], sems[buf]).wait()
        pltpu.make_async_copy(bias_hbm.at[:, pl.ds(0, CHUNK)],
                              biasring.at[buf], sems[buf]).wait()

    for j in range(NBUF):
        fire(j, j)

    def scan_step(t, llen):
        rv = idx_v[pl.ds(t * L, L)]
        m = (rv >= lo) & (rv < hi)
        cs = plsc.cumsum(m.astype(jnp.int32))
        slots = jnp.where(m, jnp.clip(llen + cs - 1, 0, B - 1), B + lanes)
        plsc.store_scatter(rlist, [slots], rv)
        plsc.store_scatter(blist, [slots], t * L + lanes)
        return llen + plsc.all_reduce_population_count(m)

    llen_splat = lax.fori_loop(0, B // L, scan_step,
                               jnp.zeros((L,), jnp.int32))
    listlen = _scalar(llen_splat)
    plsc.store_scatter(rlist, [jnp.clip(listlen + lanes, 0, B + L - 1)],
                       jnp.full((L,), BIG, jnp.int32))
    nlv = (listlen + L - 1) // L
    nseg = (listlen + BINCAP - 1) // BINCAP

    for k in range(128 // L):
        bidx[0, pl.ds(k * L, L)] = jnp.full((L,), dump, jnp.int32)

    def flush(cnt):
        def do_flush():
            pltpu.async_copy(rowstage, out_int.at[bidx.at[0]], sems[NBUF]) \
                .wait()
            for k in range(128 // L):
                bidx[0, pl.ds(k * L, L)] = jnp.full((L,), dump, jnp.int32)
        pl.when(cnt >= FLUSH_AT)(do_flush)
        return jnp.where(cnt >= FLUSH_AT, 0, cnt)

    def extract_from(src_ref, src_bias_ref, qlo, width, cnt):
        def seg_body(s, cnt):
            v0 = s * (BINCAP // L)
            v1 = jnp.minimum(nlv, v0 + BINCAP // L)

            def bin_step(v, bcnt):
                rv = rlist[pl.ds(v * L, L)]
                bv = blist[pl.ds(v * L, L)]
                m = (rv >= qlo) & (rv < qlo + width)
                cs = plsc.cumsum(m.astype(jnp.int32))
                slots = jnp.where(m, jnp.clip(bcnt + cs - 1, 0, BINCAP - 1),
                                  BINCAP + lanes)
                plsc.store_scatter(binr, [slots], rv)
                plsc.store_scatter(binb, [slots], bv)
                return bcnt + plsc.all_reduce_population_count(m)

            bcnt_splat = lax.fori_loop(v0, v1, bin_step,
                                       jnp.zeros((L,), jnp.int32))
            bincnt = _scalar(bcnt_splat)

            def grp_body(g, cnt):
                rv = binr[pl.ds(g * L, L)]
                bv = binb[pl.ds(g * L, L)]
                m = (g * L + lanes) < bincnt
                bv_safe = jnp.where(m, bv, dump)
                r_loc = jnp.clip(rv - qlo, 0, width - 1)
                slots = cnt + lanes
                for f in range(NF):
                    f16 = jnp.full((L,), f, jnp.int32)
                    val = plsc.load_gather(src_ref, [f16, r_loc])
                    plsc.store_scatter(rowstage, [slots, f16], val)
                bval = plsc.load_gather(src_bias_ref,
                                        [jnp.zeros((L,), jnp.int32), r_loc])
                plsc.store_scatter(rowstage,
                                   [slots, jnp.full((L,), NF, jnp.int32)],
                                   bval)
                plsc.store_scatter(bidx, [jnp.zeros((L,), jnp.int32), slots],
                                   bv_safe)
                cnt = cnt + jnp.minimum(L, bincnt - g * L)
                return flush(cnt)

            return lax.fori_loop(0, (bincnt + L - 1) // L, grp_body, cnt)

        return lax.fori_loop(0, nseg, seg_body, cnt)

    def ring_group(g, cnt):
        for j in range(NBUF):
            q = g * NBUF + j
            wait(j)
            qlo = (gbase + q) * CHUNK
            cnt = extract_from(staged.at[j], biasring.at[j], qlo, CHUNK, cnt)
            fire(q + NBUF, j)
        return cnt

    cnt = lax.fori_loop(0, (CPW - 1) // NBUF, ring_group, jnp.int32(0))

    wait(0)
    cnt = extract_from(staged.at[0], biasring.at[0],
                       (gbase + CPW - 1) * CHUNK, CHUNK, cnt)
    wait(1)
    is_last = wid == NW - 1
    cnt = extract_from(staged.at[1], biasring.at[1],
                       jnp.where(is_last, (NCH_FULL - 1) * CHUNK, -CHUNK),
                       CHUNK, cnt)
    wait(2)

    pltpu.sync_copy(tail_t_hbm, tailstage)
    pltpu.sync_copy(tail_b_hbm, tailbias)

    def tail_extract(cnt):
        def seg_body(s, cnt):
            v0 = s * (BINCAP // L)
            v1 = jnp.minimum(nlv, v0 + BINCAP // L)

            def bin_step(v, bcnt):
                rv = rlist[pl.ds(v * L, L)]
                bv = blist[pl.ds(v * L, L)]
                m = (rv >= TAIL0) & (rv < NR)
                cs = plsc.cumsum(m.astype(jnp.int32))
                slots = jnp.where(m, jnp.clip(bcnt + cs - 1, 0, BINCAP - 1),
                                  BINCAP + lanes)
                plsc.store_scatter(binr, [slots], rv)
                plsc.store_scatter(binb, [slots], bv)
                return bcnt + plsc.all_reduce_population_count(m)

            bcnt_splat = lax.fori_loop(v0, v1, bin_step,
                                       jnp.zeros((L,), jnp.int32))
            bincnt = _scalar(bcnt_splat)

            def grp_body(g, cnt):
                rv = binr[pl.ds(g * L, L)]
                bv = binb[pl.ds(g * L, L)]
                m = (g * L + lanes) < bincnt
                bv_safe = jnp.where(m, bv, dump)
                r_loc = jnp.clip(rv - TAIL0, 0, NTAIL - 1)
                slots = cnt + lanes
                for f in range(NF):
                    f16 = jnp.full((L,), f, jnp.int32)
                    val = plsc.load_gather(tailstage, [r_loc, f16])
                    plsc.store_scatter(rowstage, [slots, f16], val)
                bval = plsc.load_gather(tailbias, [r_loc])
                plsc.store_scatter(rowstage,
                                   [slots, jnp.full((L,), NF, jnp.int32)],
                                   bval)
                plsc.store_scatter(bidx, [jnp.zeros((L,), jnp.int32), slots],
                                   bv_safe)
                cnt = cnt + jnp.minimum(L, bincnt - g * L)
                return flush(cnt)

            return lax.fori_loop(0, (bincnt + L - 1) // L, grp_body, cnt)

        return lax.fori_loop(0, nseg, seg_body, cnt)

    cnt = tail_extract(cnt)

    def final_flush():
        pltpu.async_copy(rowstage, out_int.at[bidx.at[0]], sems[NBUF]).wait()
    pl.when(cnt > 0)(final_flush)


def _kernel_a_body(user_hbm, item_hbm, uemb_hbm, iemb_hbm,
                   ubias_hbm, ibias_hbm,
                   tail_u_hbm, tail_i_hbm, tail_ub_hbm, tail_ib_hbm,
                   u_int, v_int,
                   idx_v, rlist, blist, binr, binb, rowstage, bidx,
                   staged, biasring, tailstage, tailbias,
                   sem0, sem1, sem2, sem3):
    wid = lax.axis_index("s") * NC + lax.axis_index("c")
    dump = B + wid
    sems = [sem0, sem1, sem2, sem3]
    _pass(idx_v, rlist, blist, binr, binb, rowstage, bidx,
          staged, biasring, tailstage, tailbias, sems,
          user_hbm, uemb_hbm, ubias_hbm, tail_u_hbm, tail_ub_hbm, u_int,
          wid, dump)
    _pass(idx_v, rlist, blist, binr, binb, rowstage, bidx,
          staged, biasring, tailstage, tailbias, sems,
          item_hbm, iemb_hbm, ibias_hbm, tail_i_hbm, tail_ib_hbm, v_int,
          wid, dump)


def _kernel_b_body(u_int_hbm, v_int_hbm, out_hbm,
                   ustage, vstage, out_v, sem):
    wid = lax.axis_index("s") * NC + lax.axis_index("c")
    lanes = lax.iota(jnp.int32, L)
    base_b = wid * (B // NW)

    for c in range(B // NW // 128):
        row0 = base_b + c * 128
        buf = c % 2
        if c == 0:
            pltpu.async_copy(u_int_hbm.at[pl.ds(row0, 128)],
                             ustage.at[buf], sem)
            pltpu.async_copy(v_int_hbm.at[pl.ds(row0, 128)],
                             vstage.at[buf], sem)
        pltpu.make_async_copy(u_int_hbm.at[pl.ds(0, 128)],
                              ustage.at[buf], sem).wait()
        pltpu.make_async_copy(v_int_hbm.at[pl.ds(0, 128)],
                              vstage.at[buf], sem).wait()
        if c + 1 < B // NW // 128:
            nrow0 = base_b + (c + 1) * 128
            pltpu.async_copy(u_int_hbm.at[pl.ds(nrow0, 128)],
                             ustage.at[1 - buf], sem)
            pltpu.async_copy(v_int_hbm.at[pl.ds(nrow0, 128)],
                             vstage.at[1 - buf], sem)
        for g in range(128 // L):
            b_loc = g * L + lanes
            accs = [jnp.zeros((L,), jnp.float32) for _ in range(4)]
            for f in range(NF):
                f16 = jnp.full((L,), f, jnp.int32)
                u = plsc.load_gather(ustage.at[buf], [b_loc, f16])
                v = plsc.load_gather(vstage.at[buf], [b_loc, f16])
                accs[f % 4] = accs[f % 4] + u * v
            fb = jnp.full((L,), NF, jnp.int32)
            ub = plsc.load_gather(ustage.at[buf], [b_loc, fb])
            vb = plsc.load_gather(vstage.at[buf], [b_loc, fb])
            out_v[pl.ds(c * 128 + g * L, L)] = \
                (accs[0] + accs[1]) + (accs[2] + accs[3]) + ub + vb

    pltpu.sync_copy(out_v, out_hbm.at[pl.ds(base_b, B // NW)])


@jax.jit
def kernel(user, item, user_emb, item_emb, user_bias, item_bias):
    mesh = plsc.VectorSubcoreMesh(core_axis_name="c", subcore_axis_name="s")
    call_a = pl.kernel(
        _kernel_a_body,
        out_type=(jax.ShapeDtypeStruct((B + NW, 128), jnp.float32),
                  jax.ShapeDtypeStruct((B + NW, 128), jnp.float32)),
        mesh=mesh,
        scratch_types=[
            pltpu.VMEM((B,), jnp.int32),
            pltpu.VMEM((B + L, ), jnp.int32),
            pltpu.VMEM((B + L, ), jnp.int32),
            pltpu.VMEM((BINCAP + L,), jnp.int32),
            pltpu.VMEM((BINCAP + L,), jnp.int32),
            pltpu.VMEM((128, 128), jnp.float32),
            pltpu.VMEM((1, 128), jnp.int32),
            pltpu.VMEM((NBUF, NF, CHUNK), jnp.float32),
            pltpu.VMEM((NBUF, 1, CHUNK), jnp.float32),
            pltpu.VMEM((NTAIL, NF), jnp.float32),
            pltpu.VMEM((NTAIL,), jnp.float32),
            pltpu.SemaphoreType.DMA,
            pltpu.SemaphoreType.DMA,
            pltpu.SemaphoreType.DMA,
            pltpu.SemaphoreType.DMA,
        ],
        compiler_params=pltpu.CompilerParams(needs_layout_passes=False,
                                             use_tc_tiling_on_sc=True),
    )
    u_int, v_int = call_a(
        user, item, user_emb.T, item_emb.T, user_bias.T, item_bias.T,
        user_emb[TAIL0:], item_emb[TAIL0:],
        user_bias[TAIL0:, 0], item_bias[TAIL0:, 0])

    call_b = pl.kernel(
        _kernel_b_body,
        out_type=jax.ShapeDtypeStruct((B,), jnp.float32),
        mesh=mesh,
        scratch_types=[
            pltpu.VMEM((2, 128, 128), jnp.float32),
            pltpu.VMEM((2, 128, 128), jnp.float32),
            pltpu.VMEM((B // NW,), jnp.float32),
            pltpu.SemaphoreType.DMA,
        ],
        compiler_params=pltpu.CompilerParams(needs_layout_passes=False,
                                             use_tc_tiling_on_sc=True),
    )
    return call_b(u_int, v_int)
